# baseline (speedup 1.0000x reference)
import functools

import jax
import jax.numpy as jnp
from jax import lax
from jax.experimental import pallas as pl
from jax.experimental.pallas import tpu as pltpu
from jax.experimental.pallas import tpu_sc as plsc

N = 10000
D = 128
E = 320000
B = 16384

NC, NS = 2, 16
NW = NC * NS
EW = E // NW
CHUNK = 80
NCHUNK = EW // CHUNK
NP = 10240
SLAB = NP // NS

_mesh = plsc.VectorSubcoreMesh(core_axis_name="c", subcore_axis_name="s")


def _spmm_body(h_hbm, row_hbm, col_hbm, w_hbm, zeros_hbm, out_hbm,
               acc, idx_col, idx_row, wbuf, rows, msgs,
               gsem0, gsem1, ssem0, ssem1):
    gsems = (gsem0, gsem1)
    ssems = (ssem0, ssem1)
    c = lax.axis_index("c")
    s = lax.axis_index("s")
    wid = c * NS + s
    zs = s * SLAB
    pltpu.sync_copy(zeros_hbm.at[pl.ds(zs, SLAB)], acc.at[pl.ds(zs, SLAB)])
    plsc.subcore_barrier()

    def issue(k, b):
        sem = gsems[b]
        b = jnp.int32(b)
        base = wid * EW + k * CHUNK
        pltpu.sync_copy(col_hbm.at[pl.ds(base, CHUNK)], idx_col.at[b])
        pltpu.sync_copy(row_hbm.at[pl.ds(base, CHUNK)], idx_row.at[b])
        pltpu.sync_copy(w_hbm.at[pl.ds(base, CHUNK)], wbuf.at[b])
        pltpu.async_copy(h_hbm.at[idx_col.at[b]], rows.at[b], sem)

    def issue_guarded(k, b):
        @pl.when(k < NCHUNK)
        def _():
            issue(k, b)

    def compute(b):
        sem = gsems[b]
        b = jnp.int32(b)
        pltpu.make_async_copy(h_hbm.at[idx_col.at[b]], rows.at[b], sem).wait()

        def edge_body(e, ecarry):
            wv = plsc.load_gather(wbuf.at[b], [jnp.full((16,), e, jnp.int32)])
            for d in range(8):
                msgs[b, e, pl.ds(d * 16, 16)] = rows[b, e, pl.ds(d * 16, 16)] * wv
            return ecarry

        lax.fori_loop(jnp.int32(0), jnp.int32(CHUNK), edge_body, jnp.int32(0))

    def scatter_start(b):
        sem = ssems[b]
        b = jnp.int32(b)
        pltpu.async_copy(msgs.at[b], acc.at[idx_row.at[b]], sem, add=True)

    def scatter_wait(b):
        sem = ssems[b]
        b = jnp.int32(b)
        pltpu.make_async_copy(msgs.at[b], acc.at[idx_row.at[b]], sem).wait()

    issue(jnp.int32(0), 0)
    issue(jnp.int32(1), 1)

    def pair_body(j, carry):
        k0 = j * 2
        compute(0)
        scatter_start(0)
        compute(1)
        scatter_wait(0)
        issue_guarded(k0 + 2, 0)
        scatter_start(1)
        scatter_wait(1)
        issue_guarded(k0 + 3, 1)
        return carry

    lax.fori_loop(jnp.int32(0), jnp.int32((NCHUNK - 1) // 2), pair_body,
                  jnp.int32(0))
    compute(0)
    scatter_start(0)
    scatter_wait(0)
    plsc.subcore_barrier()
    pltpu.sync_copy(acc.at[pl.ds(zs, SLAB)], out_hbm.at[pl.ds(c * NP + zs, SLAB)])


_spmm = functools.partial(
    pl.kernel,
    out_type=jax.ShapeDtypeStruct((2 * NP, D), jnp.float32),
    mesh=_mesh,
    scratch_types=[
        pltpu.VMEM_SHARED((NP, D), jnp.float32),
        pltpu.VMEM((2, CHUNK), jnp.int32),
        pltpu.VMEM((2, CHUNK), jnp.int32),
        pltpu.VMEM((2, CHUNK), jnp.float32),
        pltpu.VMEM((2, CHUNK, D), jnp.float32),
        pltpu.VMEM((2, CHUNK, D), jnp.float32),
        pltpu.SemaphoreType.DMA,
        pltpu.SemaphoreType.DMA,
        pltpu.SemaphoreType.DMA,
        pltpu.SemaphoreType.DMA,
    ],
    compiler_params=pltpu.CompilerParams(needs_layout_passes=False),
)(_spmm_body)


GW = B // NW
GCH = 128


def _lookup_body(table_hbm, idx_hbm, out_hbm, idxv, rowsv, sem):
    c = lax.axis_index("c")
    s = lax.axis_index("s")
    wid = c * NS + s
    for j in range(GW // GCH):
        base = wid * GW + j * GCH
        pltpu.sync_copy(idx_hbm.at[pl.ds(base, GCH)], idxv)
        pltpu.async_copy(table_hbm.at[idxv], rowsv, sem).wait()
        pltpu.sync_copy(rowsv, out_hbm.at[pl.ds(base, GCH)])


_lookup = functools.partial(
    pl.kernel,
    out_type=jax.ShapeDtypeStruct((B, D), jnp.float32),
    mesh=_mesh,
    scratch_types=[
        pltpu.VMEM((GCH,), jnp.int32),
        pltpu.VMEM((GCH, D), jnp.float32),
        pltpu.SemaphoreType.DMA,
    ],
)(_lookup_body)


def _mm_body(x_ref, w_ref, b_ref, o_ref):
    o_ref[...] = (jnp.dot(x_ref[...], w_ref[...],
                          preferred_element_type=jnp.float32) + b_ref[...])


def _mm(x, w, b):
    return pl.pallas_call(
        _mm_body,
        out_shape=jax.ShapeDtypeStruct((N, D), jnp.float32),
    )(x, w, b.reshape(1, D))


def _post_body(has_res, has_mm, *refs):
    refs = list(refs)
    p_ref, g_ref, be_ref = refs[:3]
    pos = 3
    res_ref = refs[pos] if has_res else None
    pos += int(has_res)
    if has_mm:
        w_ref, b_ref = refs[pos:pos + 2]
        pos += 2
    e_ref = refs[pos]
    pv = p_ref[...]
    h = jax.nn.relu(pv[:N, :] + pv[NP:NP + N, :])
    mu = jnp.mean(h, axis=1, keepdims=True)
    var = jnp.mean((h - mu) * (h - mu), axis=1, keepdims=True)
    e = (h - mu) * lax.rsqrt(var + 1e-5) * g_ref[...] + be_ref[...]
    if has_res:
        e = e + res_ref[...]
    e_ref[...] = e
    if has_mm:
        refs[pos + 1][...] = (jnp.dot(e, w_ref[...],
                                      preferred_element_type=jnp.float32)
                              + b_ref[...])


def _post(p, g, be, res=None, w=None, b=None):
    has_res = res is not None
    has_mm = w is not None
    args = [p, g.reshape(1, D), be.reshape(1, D)]
    if has_res:
        args.append(res)
    if has_mm:
        args.extend([w, b.reshape(1, D)])
    out_shape = [jax.ShapeDtypeStruct((N, D), jnp.float32)]
    if has_mm:
        out_shape.append(jax.ShapeDtypeStruct((N, D), jnp.float32))
    out = pl.pallas_call(
        functools.partial(_post_body, has_res, has_mm),
        out_shape=out_shape,
    )(*args)
    return out if has_mm else out[0]


def kernel(x, edge_row, edge_col, edge_weight, embed,
           W1, b1, W2, b2, W3, b3, g1, be1, g2, be2, g3, be3):
    idx = (x - 1).astype(jnp.int32)
    er = edge_row.astype(jnp.int32)
    ec = edge_col.astype(jnp.int32)
    emb = embed.astype(jnp.float32)
    zeros = jnp.zeros((NP, D), jnp.float32)

    h = _mm(emb, W1, b1)
    p = _spmm(h, er, ec, edge_weight, zeros)
    e1, h = _post(p, g1, be1, None, W2, b2)
    p = _spmm(h, er, ec, edge_weight, zeros)
    e2, h = _post(p, g2, be2, e1, W3, b3)
    p = _spmm(h, er, ec, edge_weight, zeros)
    e3 = _post(p, g3, be3, e2)

    out = _lookup(e3, idx)
    recon_loss = jnp.zeros((1,), dtype=jnp.float32)
    return (out, recon_loss)


# speedup vs baseline: 1.7477x; 1.7477x over previous
import functools

import jax
import jax.numpy as jnp
from jax import lax
from jax.experimental import pallas as pl
from jax.experimental.pallas import tpu as pltpu
from jax.experimental.pallas import tpu_sc as plsc

N = 10000
D = 128
E = 320000
B = 16384

NC, NS = 2, 16
NW = NC * NS
EW = E // NW
CHUNK = 80
NCHUNK = EW // CHUNK
NP = 10240
SLAB = NP // NS

_mesh = plsc.VectorSubcoreMesh(core_axis_name="c", subcore_axis_name="s")


def _spmm_body(h_hbm, row_hbm, col_hbm, w_hbm, zeros_hbm, out_hbm,
               acc, idx_col, idx_row, wbuf, rows,
               gsem0, gsem1, ssem0, ssem1):
    gsems = (gsem0, gsem1)
    ssems = (ssem0, ssem1)
    c = lax.axis_index("c")
    s = lax.axis_index("s")
    wid = c * NS + s
    zs = s * SLAB
    pltpu.sync_copy(zeros_hbm.at[pl.ds(zs, SLAB)], acc.at[pl.ds(zs, SLAB)])
    plsc.subcore_barrier()

    def issue(k, b):
        sem = gsems[b]
        b = jnp.int32(b)
        base = wid * EW + k * CHUNK
        pltpu.sync_copy(col_hbm.at[pl.ds(base, CHUNK)], idx_col.at[b])
        pltpu.sync_copy(row_hbm.at[pl.ds(base, CHUNK)], idx_row.at[b])
        pltpu.sync_copy(w_hbm.at[pl.ds(base, CHUNK)], wbuf.at[b])
        pltpu.async_copy(h_hbm.at[idx_col.at[b]], rows.at[b], sem)

    def issue_guarded(k, b):
        @pl.when(k < NCHUNK)
        def _():
            issue(k, b)

    def compute(b):
        sem = gsems[b]
        b = jnp.int32(b)
        pltpu.make_async_copy(h_hbm.at[idx_col.at[b]], rows.at[b], sem).wait()

        def edge_body(e, ecarry):
            wv = plsc.load_gather(wbuf.at[b], [jnp.full((16,), e, jnp.int32)])
            for d in range(8):
                rows[b, e, pl.ds(d * 16, 16)] = rows[b, e, pl.ds(d * 16, 16)] * wv
            return ecarry

        lax.fori_loop(jnp.int32(0), jnp.int32(CHUNK), edge_body, jnp.int32(0))

    def scatter_start(b):
        sem = ssems[b]
        b = jnp.int32(b)
        pltpu.async_copy(rows.at[b], acc.at[idx_row.at[b]], sem, add=True)

    def scatter_wait(b):
        sem = ssems[b]
        b = jnp.int32(b)
        pltpu.make_async_copy(rows.at[b], acc.at[idx_row.at[b]], sem).wait()

    issue(jnp.int32(0), 0)
    issue(jnp.int32(1), 1)

    def pair_body(j, carry):
        k0 = j * 2
        compute(0)
        scatter_start(0)
        compute(1)
        scatter_wait(0)
        issue_guarded(k0 + 2, 0)
        scatter_start(1)
        scatter_wait(1)
        issue_guarded(k0 + 3, 1)
        return carry

    lax.fori_loop(jnp.int32(0), jnp.int32((NCHUNK - 1) // 2), pair_body,
                  jnp.int32(0))
    compute(0)
    scatter_start(0)
    scatter_wait(0)
    plsc.subcore_barrier()
    pltpu.sync_copy(acc.at[pl.ds(zs, SLAB)], out_hbm.at[pl.ds(c * NP + zs, SLAB)])


_spmm = functools.partial(
    pl.kernel,
    out_type=jax.ShapeDtypeStruct((2 * NP, D), jnp.float32),
    mesh=_mesh,
    scratch_types=[
        pltpu.VMEM_SHARED((NP, D), jnp.float32),
        pltpu.VMEM((2, CHUNK), jnp.int32),
        pltpu.VMEM((2, CHUNK), jnp.int32),
        pltpu.VMEM((2, CHUNK), jnp.float32),
        pltpu.VMEM((2, CHUNK, D), jnp.float32),
        pltpu.SemaphoreType.DMA,
        pltpu.SemaphoreType.DMA,
        pltpu.SemaphoreType.DMA,
        pltpu.SemaphoreType.DMA,
    ],
    compiler_params=pltpu.CompilerParams(needs_layout_passes=False),
)(_spmm_body)


GW = B // NW
GCH = 128


def _lookup_body(table_hbm, idx_hbm, out_hbm, idxv, rowsv, sem):
    c = lax.axis_index("c")
    s = lax.axis_index("s")
    wid = c * NS + s
    for j in range(GW // GCH):
        base = wid * GW + j * GCH
        pltpu.sync_copy(idx_hbm.at[pl.ds(base, GCH)], idxv)
        pltpu.async_copy(table_hbm.at[idxv], rowsv, sem).wait()
        pltpu.sync_copy(rowsv, out_hbm.at[pl.ds(base, GCH)])


_lookup = functools.partial(
    pl.kernel,
    out_type=jax.ShapeDtypeStruct((B, D), jnp.float32),
    mesh=_mesh,
    scratch_types=[
        pltpu.VMEM((GCH,), jnp.int32),
        pltpu.VMEM((GCH, D), jnp.float32),
        pltpu.SemaphoreType.DMA,
    ],
)(_lookup_body)


def _mm_body(x_ref, w_ref, b_ref, o_ref):
    o_ref[...] = (jnp.dot(x_ref[...], w_ref[...],
                          preferred_element_type=jnp.float32) + b_ref[...])


def _mm(x, w, b):
    return pl.pallas_call(
        _mm_body,
        out_shape=jax.ShapeDtypeStruct((N, D), jnp.float32),
    )(x, w, b.reshape(1, D))


def _post_body(has_res, has_mm, *refs):
    refs = list(refs)
    p_ref, g_ref, be_ref = refs[:3]
    pos = 3
    res_ref = refs[pos] if has_res else None
    pos += int(has_res)
    if has_mm:
        w_ref, b_ref = refs[pos:pos + 2]
        pos += 2
    e_ref = refs[pos]
    pv = p_ref[...]
    h = jax.nn.relu(pv[:N, :] + pv[NP:NP + N, :])
    mu = jnp.mean(h, axis=1, keepdims=True)
    var = jnp.mean((h - mu) * (h - mu), axis=1, keepdims=True)
    e = (h - mu) * lax.rsqrt(var + 1e-5) * g_ref[...] + be_ref[...]
    if has_res:
        e = e + res_ref[...]
    e_ref[...] = e
    if has_mm:
        refs[pos + 1][...] = (jnp.dot(e, w_ref[...],
                                      preferred_element_type=jnp.float32)
                              + b_ref[...])


def _post(p, g, be, res=None, w=None, b=None):
    has_res = res is not None
    has_mm = w is not None
    args = [p, g.reshape(1, D), be.reshape(1, D)]
    if has_res:
        args.append(res)
    if has_mm:
        args.extend([w, b.reshape(1, D)])
    out_shape = [jax.ShapeDtypeStruct((N, D), jnp.float32)]
    if has_mm:
        out_shape.append(jax.ShapeDtypeStruct((N, D), jnp.float32))
    out = pl.pallas_call(
        functools.partial(_post_body, has_res, has_mm),
        out_shape=out_shape,
    )(*args)
    return out if has_mm else out[0]


def kernel(x, edge_row, edge_col, edge_weight, embed,
           W1, b1, W2, b2, W3, b3, g1, be1, g2, be2, g3, be3):
    idx = (x - 1).astype(jnp.int32)
    er = edge_row.astype(jnp.int32)
    ec = edge_col.astype(jnp.int32)
    emb = embed.astype(jnp.float32)
    zeros = jnp.zeros((NP, D), jnp.float32)

    h = _mm(emb, W1, b1)
    p = _spmm(h, er, ec, edge_weight, zeros)
    e1, h = _post(p, g1, be1, None, W2, b2)
    p = _spmm(h, er, ec, edge_weight, zeros)
    e2, h = _post(p, g2, be2, e1, W3, b3)
    p = _spmm(h, er, ec, edge_weight, zeros)
    e3 = _post(p, g3, be3, e2)

    out = _lookup(e3, idx)
    recon_loss = jnp.zeros((1,), dtype=jnp.float32)
    return (out, recon_loss)


# speedup vs baseline: 1.8628x; 1.0658x over previous
import functools

import jax
import jax.numpy as jnp
from jax import lax
from jax.experimental import pallas as pl
from jax.experimental.pallas import tpu as pltpu
from jax.experimental.pallas import tpu_sc as plsc

N = 10000
D = 128
E = 320000
B = 16384

NC, NS = 2, 16
NW = NC * NS
EW = E // NW
CHUNK = 80
NCHUNK = EW // CHUNK
NP = 10240
SLAB = NP // NS

_mesh = plsc.VectorSubcoreMesh(core_axis_name="c", subcore_axis_name="s")


def _spmm_body(h_hbm, row_hbm, col_hbm, w_hbm, zeros_hbm, out_hbm,
               acc, idx_col, idx_row, wbuf, rows,
               gsem0, gsem1, ssem0, ssem1):
    gsems = (gsem0, gsem1)
    ssems = (ssem0, ssem1)
    c = lax.axis_index("c")
    s = lax.axis_index("s")
    wid = c * NS + s
    zs = s * SLAB
    pltpu.sync_copy(zeros_hbm.at[pl.ds(zs, SLAB)], acc.at[pl.ds(zs, SLAB)])
    plsc.subcore_barrier()

    def issue(k, b):
        sem = gsems[b]
        b = jnp.int32(b)
        base = wid * EW + k * CHUNK
        pltpu.sync_copy(col_hbm.at[pl.ds(base, CHUNK)], idx_col.at[b])
        pltpu.sync_copy(row_hbm.at[pl.ds(base, CHUNK)], idx_row.at[b])
        pltpu.sync_copy(w_hbm.at[pl.ds(base, CHUNK)], wbuf.at[b])
        pltpu.async_copy(h_hbm.at[idx_col.at[b]], rows.at[b], sem)

    def issue_guarded(k, b):
        @pl.when(k < NCHUNK)
        def _():
            issue(k, b)

    def compute(b):
        sem = gsems[b]
        b = jnp.int32(b)
        pltpu.make_async_copy(h_hbm.at[idx_col.at[b]], rows.at[b], sem).wait()

        def edge_body(e4, ecarry):
            e0 = e4 * 4
            wvs = [plsc.load_gather(wbuf.at[b],
                                    [jnp.full((16,), e0 + u, jnp.int32)])
                   for u in range(4)]
            for d in range(8):
                for u in range(4):
                    rows[b, e0 + u, pl.ds(d * 16, 16)] = (
                        rows[b, e0 + u, pl.ds(d * 16, 16)] * wvs[u])
            return ecarry

        lax.fori_loop(jnp.int32(0), jnp.int32(CHUNK // 4), edge_body,
                      jnp.int32(0))

    def scatter_start(b):
        sem = ssems[b]
        b = jnp.int32(b)
        pltpu.async_copy(rows.at[b], acc.at[idx_row.at[b]], sem, add=True)

    def scatter_wait(b):
        sem = ssems[b]
        b = jnp.int32(b)
        pltpu.make_async_copy(rows.at[b], acc.at[idx_row.at[b]], sem).wait()

    issue(jnp.int32(0), 0)
    issue(jnp.int32(1), 1)

    def pair_body(j, carry):
        k0 = j * 2
        compute(0)
        scatter_start(0)
        compute(1)
        scatter_wait(0)
        issue_guarded(k0 + 2, 0)
        scatter_start(1)
        scatter_wait(1)
        issue_guarded(k0 + 3, 1)
        return carry

    lax.fori_loop(jnp.int32(0), jnp.int32((NCHUNK - 1) // 2), pair_body,
                  jnp.int32(0))
    compute(0)
    scatter_start(0)
    scatter_wait(0)
    plsc.subcore_barrier()
    pltpu.sync_copy(acc.at[pl.ds(zs, SLAB)], out_hbm.at[pl.ds(c * NP + zs, SLAB)])


_spmm = functools.partial(
    pl.kernel,
    out_type=jax.ShapeDtypeStruct((2 * NP, D), jnp.float32),
    mesh=_mesh,
    scratch_types=[
        pltpu.VMEM_SHARED((NP, D), jnp.float32),
        pltpu.VMEM((2, CHUNK), jnp.int32),
        pltpu.VMEM((2, CHUNK), jnp.int32),
        pltpu.VMEM((2, CHUNK), jnp.float32),
        pltpu.VMEM((2, CHUNK, D), jnp.float32),
        pltpu.SemaphoreType.DMA,
        pltpu.SemaphoreType.DMA,
        pltpu.SemaphoreType.DMA,
        pltpu.SemaphoreType.DMA,
    ],
    compiler_params=pltpu.CompilerParams(needs_layout_passes=False),
)(_spmm_body)


GW = B // NW
GCH = 128


def _lookup_body(table_hbm, idx_hbm, out_hbm, idxv, rowsv, sem):
    c = lax.axis_index("c")
    s = lax.axis_index("s")
    wid = c * NS + s
    for j in range(GW // GCH):
        base = wid * GW + j * GCH
        pltpu.sync_copy(idx_hbm.at[pl.ds(base, GCH)], idxv)
        pltpu.async_copy(table_hbm.at[idxv], rowsv, sem).wait()
        pltpu.sync_copy(rowsv, out_hbm.at[pl.ds(base, GCH)])


_lookup = functools.partial(
    pl.kernel,
    out_type=jax.ShapeDtypeStruct((B, D), jnp.float32),
    mesh=_mesh,
    scratch_types=[
        pltpu.VMEM((GCH,), jnp.int32),
        pltpu.VMEM((GCH, D), jnp.float32),
        pltpu.SemaphoreType.DMA,
    ],
)(_lookup_body)


def _mm_body(x_ref, w_ref, b_ref, o_ref):
    o_ref[...] = (jnp.dot(x_ref[...], w_ref[...],
                          preferred_element_type=jnp.float32) + b_ref[...])


def _mm(x, w, b):
    return pl.pallas_call(
        _mm_body,
        out_shape=jax.ShapeDtypeStruct((N, D), jnp.float32),
    )(x, w, b.reshape(1, D))


def _post_body(has_res, has_mm, *refs):
    refs = list(refs)
    p_ref, g_ref, be_ref = refs[:3]
    pos = 3
    res_ref = refs[pos] if has_res else None
    pos += int(has_res)
    if has_mm:
        w_ref, b_ref = refs[pos:pos + 2]
        pos += 2
    e_ref = refs[pos]
    pv = p_ref[...]
    h = jax.nn.relu(pv[:N, :] + pv[NP:NP + N, :])
    mu = jnp.mean(h, axis=1, keepdims=True)
    var = jnp.mean((h - mu) * (h - mu), axis=1, keepdims=True)
    e = (h - mu) * lax.rsqrt(var + 1e-5) * g_ref[...] + be_ref[...]
    if has_res:
        e = e + res_ref[...]
    e_ref[...] = e
    if has_mm:
        refs[pos + 1][...] = (jnp.dot(e, w_ref[...],
                                      preferred_element_type=jnp.float32)
                              + b_ref[...])


def _post(p, g, be, res=None, w=None, b=None):
    has_res = res is not None
    has_mm = w is not None
    args = [p, g.reshape(1, D), be.reshape(1, D)]
    if has_res:
        args.append(res)
    if has_mm:
        args.extend([w, b.reshape(1, D)])
    out_shape = [jax.ShapeDtypeStruct((N, D), jnp.float32)]
    if has_mm:
        out_shape.append(jax.ShapeDtypeStruct((N, D), jnp.float32))
    out = pl.pallas_call(
        functools.partial(_post_body, has_res, has_mm),
        out_shape=out_shape,
    )(*args)
    return out if has_mm else out[0]


def kernel(x, edge_row, edge_col, edge_weight, embed,
           W1, b1, W2, b2, W3, b3, g1, be1, g2, be2, g3, be3):
    idx = (x - 1).astype(jnp.int32)
    er = edge_row.astype(jnp.int32)
    ec = edge_col.astype(jnp.int32)
    emb = embed.astype(jnp.float32)
    zeros = jnp.zeros((NP, D), jnp.float32)

    h = _mm(emb, W1, b1)
    p = _spmm(h, er, ec, edge_weight, zeros)
    e1, h = _post(p, g1, be1, None, W2, b2)
    p = _spmm(h, er, ec, edge_weight, zeros)
    e2, h = _post(p, g2, be2, e1, W3, b3)
    p = _spmm(h, er, ec, edge_weight, zeros)
    e3 = _post(p, g3, be3, e2)

    out = _lookup(e3, idx)
    recon_loss = jnp.zeros((1,), dtype=jnp.float32)
    return (out, recon_loss)


# speedup vs baseline: 2.5541x; 1.3711x over previous
import functools

import jax
import jax.numpy as jnp
from jax import lax
from jax.experimental import pallas as pl
from jax.experimental.pallas import tpu as pltpu
from jax.experimental.pallas import tpu_sc as plsc

N = 10000
D = 128
E = 320000
B = 16384

NC, NS = 2, 16
NW = NC * NS
EW = E // NW
CHUNK = 80
NCHUNK = EW // CHUNK
NP = 10240
SLAB = NP // NS

_mesh = plsc.VectorSubcoreMesh(core_axis_name="c", subcore_axis_name="s")


def _spmm_body(h_hbm, epack_hbm, zeros_hbm, out_hbm,
               acc, ep0, ep1, rows0, rows1,
               gsem0, gsem1, ssem0, ssem1):
    gsems = (gsem0, gsem1)
    ssems = (ssem0, ssem1)
    epbufs = (ep0, ep1)
    rowbufs = (rows0, rows1)
    c = lax.axis_index("c")
    s = lax.axis_index("s")
    wid = c * NS + s
    zs = s * SLAB
    pltpu.sync_copy(zeros_hbm.at[pl.ds(zs, SLAB)], acc.at[pl.ds(zs, SLAB)])
    plsc.subcore_barrier()

    def icol(b):
        return epbufs[b].at[jnp.int32(0)]

    def irow(b):
        return epbufs[b].at[jnp.int32(1)]

    def issue(k, b):
        g = wid * NCHUNK + k
        pltpu.sync_copy(epack_hbm.at[g], epbufs[b])
        pltpu.async_copy(h_hbm.at[icol(b)], rowbufs[b], gsems[b])

    def issue_guarded(k, b):
        @pl.when(k < NCHUNK)
        def _():
            issue(k, b)

    def compute(b):
        pltpu.make_async_copy(h_hbm.at[icol(b)], rowbufs[b], gsems[b]).wait()
        rows = rowbufs[b]
        wrow = epbufs[b].at[jnp.int32(2)]

        def edge_body(e4, ecarry):
            e0 = e4 * 4
            wvs = [plsc.bitcast(
                       plsc.load_gather(wrow,
                                        [jnp.full((16,), e0 + u, jnp.int32)]),
                       jnp.float32)
                   for u in range(4)]
            for d in range(8):
                for u in range(4):
                    rows[e0 + u, pl.ds(d * 16, 16)] = (
                        rows[e0 + u, pl.ds(d * 16, 16)] * wvs[u])
            return ecarry

        lax.fori_loop(jnp.int32(0), jnp.int32(CHUNK // 4), edge_body,
                      jnp.int32(0))

    def scatter_start(b):
        pltpu.async_copy(rowbufs[b], acc.at[irow(b)], ssems[b], add=True)

    def scatter_wait(b):
        pltpu.make_async_copy(rowbufs[b], acc.at[irow(b)], ssems[b]).wait()

    issue(jnp.int32(0), 0)
    issue(jnp.int32(1), 1)

    def pair_body(j, carry):
        k0 = j * 2
        compute(0)
        scatter_start(0)
        compute(1)
        scatter_wait(0)
        issue_guarded(k0 + 2, 0)
        scatter_start(1)
        scatter_wait(1)
        issue_guarded(k0 + 3, 1)
        return carry

    lax.fori_loop(jnp.int32(0), jnp.int32((NCHUNK - 1) // 2), pair_body,
                  jnp.int32(0))
    compute(0)
    scatter_start(0)
    scatter_wait(0)
    plsc.subcore_barrier()
    pltpu.sync_copy(acc.at[pl.ds(zs, SLAB)], out_hbm.at[pl.ds(c * NP + zs, SLAB)])


_spmm = functools.partial(
    pl.kernel,
    out_type=jax.ShapeDtypeStruct((2 * NP, D), jnp.float32),
    mesh=_mesh,
    scratch_types=[
        pltpu.VMEM_SHARED((NP, D), jnp.float32),
        pltpu.VMEM((3, CHUNK), jnp.int32),
        pltpu.VMEM((3, CHUNK), jnp.int32),
        pltpu.VMEM((CHUNK, D), jnp.float32),
        pltpu.VMEM((CHUNK, D), jnp.float32),
        pltpu.SemaphoreType.DMA,
        pltpu.SemaphoreType.DMA,
        pltpu.SemaphoreType.DMA,
        pltpu.SemaphoreType.DMA,
    ],
    compiler_params=pltpu.CompilerParams(needs_layout_passes=False),
)(_spmm_body)


GW = B // NW
GCH = 128


def _lookup_body(table_hbm, idx_hbm, out_hbm, idxv, rowsv, sem):
    c = lax.axis_index("c")
    s = lax.axis_index("s")
    wid = c * NS + s
    for j in range(GW // GCH):
        base = wid * GW + j * GCH
        pltpu.sync_copy(idx_hbm.at[pl.ds(base, GCH)], idxv)
        pltpu.async_copy(table_hbm.at[idxv], rowsv, sem).wait()
        pltpu.sync_copy(rowsv, out_hbm.at[pl.ds(base, GCH)])


_lookup = functools.partial(
    pl.kernel,
    out_type=jax.ShapeDtypeStruct((B, D), jnp.float32),
    mesh=_mesh,
    scratch_types=[
        pltpu.VMEM((GCH,), jnp.int32),
        pltpu.VMEM((GCH, D), jnp.float32),
        pltpu.SemaphoreType.DMA,
    ],
)(_lookup_body)


def _mm_body(x_ref, w_ref, b_ref, o_ref):
    o_ref[...] = (jnp.dot(x_ref[...], w_ref[...],
                          preferred_element_type=jnp.float32) + b_ref[...])


def _mm(x, w, b):
    return pl.pallas_call(
        _mm_body,
        out_shape=jax.ShapeDtypeStruct((N, D), jnp.float32),
    )(x, w, b.reshape(1, D))


def _post_body(has_res, has_mm, *refs):
    refs = list(refs)
    p_ref, g_ref, be_ref = refs[:3]
    pos = 3
    res_ref = refs[pos] if has_res else None
    pos += int(has_res)
    if has_mm:
        w_ref, b_ref = refs[pos:pos + 2]
        pos += 2
    e_ref = refs[pos]
    pv = p_ref[...]
    h = jax.nn.relu(pv[:N, :] + pv[NP:NP + N, :])
    mu = jnp.mean(h, axis=1, keepdims=True)
    var = jnp.mean((h - mu) * (h - mu), axis=1, keepdims=True)
    e = (h - mu) * lax.rsqrt(var + 1e-5) * g_ref[...] + be_ref[...]
    if has_res:
        e = e + res_ref[...]
    e_ref[...] = e
    if has_mm:
        refs[pos + 1][...] = (jnp.dot(e, w_ref[...],
                                      preferred_element_type=jnp.float32)
                              + b_ref[...])


def _post(p, g, be, res=None, w=None, b=None):
    has_res = res is not None
    has_mm = w is not None
    args = [p, g.reshape(1, D), be.reshape(1, D)]
    if has_res:
        args.append(res)
    if has_mm:
        args.extend([w, b.reshape(1, D)])
    out_shape = [jax.ShapeDtypeStruct((N, D), jnp.float32)]
    if has_mm:
        out_shape.append(jax.ShapeDtypeStruct((N, D), jnp.float32))
    out = pl.pallas_call(
        functools.partial(_post_body, has_res, has_mm),
        out_shape=out_shape,
    )(*args)
    return out if has_mm else out[0]


def kernel(x, edge_row, edge_col, edge_weight, embed,
           W1, b1, W2, b2, W3, b3, g1, be1, g2, be2, g3, be3):
    idx = (x - 1).astype(jnp.int32)
    er = edge_row.astype(jnp.int32)
    ec = edge_col.astype(jnp.int32)
    wbits = lax.bitcast_convert_type(edge_weight.astype(jnp.float32),
                                     jnp.int32)
    epack = jnp.stack([ec.reshape(-1, CHUNK), er.reshape(-1, CHUNK),
                       wbits.reshape(-1, CHUNK)], axis=1)
    emb = embed.astype(jnp.float32)
    zeros = jnp.zeros((NP, D), jnp.float32)

    h = _mm(emb, W1, b1)
    p = _spmm(h, epack, zeros)
    e1, h = _post(p, g1, be1, None, W2, b2)
    p = _spmm(h, epack, zeros)
    e2, h = _post(p, g2, be2, e1, W3, b3)
    p = _spmm(h, epack, zeros)
    e3 = _post(p, g3, be3, e2)

    out = _lookup(e3, idx)
    recon_loss = jnp.zeros((1,), dtype=jnp.float32)
    return (out, recon_loss)


# speedup vs baseline: 2.5672x; 1.0051x over previous
import functools

import jax
import jax.numpy as jnp
from jax import lax
from jax.experimental import pallas as pl
from jax.experimental.pallas import tpu as pltpu
from jax.experimental.pallas import tpu_sc as plsc

N = 10000
D = 128
E = 320000
B = 16384

NC, NS = 2, 16
NW = NC * NS
EW = E // NW
CHUNK = 80
NCHUNK = EW // CHUNK
NP = 10240
SLAB = NP // NS

_mesh = plsc.VectorSubcoreMesh(core_axis_name="c", subcore_axis_name="s")


def _spmm_body(h_hbm, epack_hbm, zeros_hbm, out_hbm,
               acc, ep0, ep1, rows0, rows1,
               gsem0, gsem1, ssem0, ssem1):
    gsems = (gsem0, gsem1)
    ssems = (ssem0, ssem1)
    epbufs = (ep0, ep1)
    rowbufs = (rows0, rows1)
    c = lax.axis_index("c")
    s = lax.axis_index("s")
    wid = c * NS + s
    zs = s * SLAB
    pltpu.sync_copy(zeros_hbm.at[pl.ds(zs, SLAB)], acc.at[pl.ds(zs, SLAB)])
    plsc.subcore_barrier()

    def icol(b):
        return epbufs[b].at[jnp.int32(0)]

    def irow(b):
        return epbufs[b].at[jnp.int32(1)]

    def issue(k, b):
        g = wid * NCHUNK + k
        pltpu.sync_copy(epack_hbm.at[g], epbufs[b])
        pltpu.async_copy(h_hbm.at[icol(b)], rowbufs[b], gsems[b])

    def issue_guarded(k, b):
        @pl.when(k < NCHUNK)
        def _():
            issue(k, b)

    def compute(b):
        pltpu.make_async_copy(h_hbm.at[icol(b)], rowbufs[b], gsems[b]).wait()
        rows = rowbufs[b]
        wrow = epbufs[b].at[jnp.int32(2)]

        def edge_body(e8, ecarry):
            e0 = e8 * 8
            wvs = [plsc.bitcast(
                       plsc.load_gather(wrow,
                                        [jnp.full((16,), e0 + u, jnp.int32)]),
                       jnp.float32)
                   for u in range(8)]
            for d in range(8):
                for u in range(8):
                    rows[e0 + u, pl.ds(d * 16, 16)] = (
                        rows[e0 + u, pl.ds(d * 16, 16)] * wvs[u])
            return ecarry

        lax.fori_loop(jnp.int32(0), jnp.int32(CHUNK // 8), edge_body,
                      jnp.int32(0))

    def scatter_start(b):
        pltpu.async_copy(rowbufs[b], acc.at[irow(b)], ssems[b], add=True)

    def scatter_wait(b):
        pltpu.make_async_copy(rowbufs[b], acc.at[irow(b)], ssems[b]).wait()

    issue(jnp.int32(0), 0)
    issue(jnp.int32(1), 1)

    def pair_body(j, carry):
        k0 = j * 2
        compute(0)
        scatter_start(0)
        compute(1)
        scatter_wait(0)
        issue_guarded(k0 + 2, 0)
        scatter_start(1)
        scatter_wait(1)
        issue_guarded(k0 + 3, 1)
        return carry

    lax.fori_loop(jnp.int32(0), jnp.int32((NCHUNK - 1) // 2), pair_body,
                  jnp.int32(0))
    compute(0)
    scatter_start(0)
    scatter_wait(0)
    plsc.subcore_barrier()
    pltpu.sync_copy(acc.at[pl.ds(zs, SLAB)], out_hbm.at[pl.ds(c * NP + zs, SLAB)])


_spmm = functools.partial(
    pl.kernel,
    out_type=jax.ShapeDtypeStruct((2 * NP, D), jnp.float32),
    mesh=_mesh,
    scratch_types=[
        pltpu.VMEM_SHARED((NP, D), jnp.float32),
        pltpu.VMEM((3, CHUNK), jnp.int32),
        pltpu.VMEM((3, CHUNK), jnp.int32),
        pltpu.VMEM((CHUNK, D), jnp.float32),
        pltpu.VMEM((CHUNK, D), jnp.float32),
        pltpu.SemaphoreType.DMA,
        pltpu.SemaphoreType.DMA,
        pltpu.SemaphoreType.DMA,
        pltpu.SemaphoreType.DMA,
    ],
    compiler_params=pltpu.CompilerParams(needs_layout_passes=False),
)(_spmm_body)


GW = B // NW
GCH = 128


def _lookup_body(table_hbm, idx_hbm, out_hbm, idxv, rowsv, sem):
    c = lax.axis_index("c")
    s = lax.axis_index("s")
    wid = c * NS + s
    for j in range(GW // GCH):
        base = wid * GW + j * GCH
        pltpu.sync_copy(idx_hbm.at[pl.ds(base, GCH)], idxv)
        pltpu.async_copy(table_hbm.at[idxv], rowsv, sem).wait()
        pltpu.sync_copy(rowsv, out_hbm.at[pl.ds(base, GCH)])


_lookup = functools.partial(
    pl.kernel,
    out_type=jax.ShapeDtypeStruct((B, D), jnp.float32),
    mesh=_mesh,
    scratch_types=[
        pltpu.VMEM((GCH,), jnp.int32),
        pltpu.VMEM((GCH, D), jnp.float32),
        pltpu.SemaphoreType.DMA,
    ],
)(_lookup_body)


def _mm_body(x_ref, w_ref, b_ref, o_ref):
    o_ref[...] = (jnp.dot(x_ref[...], w_ref[...],
                          preferred_element_type=jnp.float32) + b_ref[...])


def _mm(x, w, b):
    return pl.pallas_call(
        _mm_body,
        out_shape=jax.ShapeDtypeStruct((N, D), jnp.float32),
    )(x, w, b.reshape(1, D))


def _post_body(has_res, has_mm, *refs):
    refs = list(refs)
    p_ref, g_ref, be_ref = refs[:3]
    pos = 3
    res_ref = refs[pos] if has_res else None
    pos += int(has_res)
    if has_mm:
        w_ref, b_ref = refs[pos:pos + 2]
        pos += 2
    e_ref = refs[pos]
    pv = p_ref[...]
    h = jax.nn.relu(pv[:N, :] + pv[NP:NP + N, :])
    mu = jnp.mean(h, axis=1, keepdims=True)
    var = jnp.mean((h - mu) * (h - mu), axis=1, keepdims=True)
    e = (h - mu) * lax.rsqrt(var + 1e-5) * g_ref[...] + be_ref[...]
    if has_res:
        e = e + res_ref[...]
    e_ref[...] = e
    if has_mm:
        refs[pos + 1][...] = (jnp.dot(e, w_ref[...],
                                      preferred_element_type=jnp.float32)
                              + b_ref[...])


def _post(p, g, be, res=None, w=None, b=None):
    has_res = res is not None
    has_mm = w is not None
    args = [p, g.reshape(1, D), be.reshape(1, D)]
    if has_res:
        args.append(res)
    if has_mm:
        args.extend([w, b.reshape(1, D)])
    out_shape = [jax.ShapeDtypeStruct((N, D), jnp.float32)]
    if has_mm:
        out_shape.append(jax.ShapeDtypeStruct((N, D), jnp.float32))
    out = pl.pallas_call(
        functools.partial(_post_body, has_res, has_mm),
        out_shape=out_shape,
    )(*args)
    return out if has_mm else out[0]


def kernel(x, edge_row, edge_col, edge_weight, embed,
           W1, b1, W2, b2, W3, b3, g1, be1, g2, be2, g3, be3):
    idx = (x - 1).astype(jnp.int32)
    er = edge_row.astype(jnp.int32)
    ec = edge_col.astype(jnp.int32)
    wbits = lax.bitcast_convert_type(edge_weight.astype(jnp.float32),
                                     jnp.int32)
    epack = jnp.stack([ec.reshape(-1, CHUNK), er.reshape(-1, CHUNK),
                       wbits.reshape(-1, CHUNK)], axis=1)
    emb = embed.astype(jnp.float32)
    zeros = jnp.zeros((NP, D), jnp.float32)

    h = _mm(emb, W1, b1)
    p = _spmm(h, epack, zeros)
    e1, h = _post(p, g1, be1, None, W2, b2)
    p = _spmm(h, epack, zeros)
    e2, h = _post(p, g2, be2, e1, W3, b3)
    p = _spmm(h, epack, zeros)
    e3 = _post(p, g3, be3, e2)

    out = _lookup(e3, idx)
    recon_loss = jnp.zeros((1,), dtype=jnp.float32)
    return (out, recon_loss)
